# Initial kernel scaffold; baseline (speedup 1.0000x reference)
import jax
import jax.numpy as jnp
from jax.experimental import pallas as pl


def kernel(x, edge_index, W_l, b_l, W_r, gamma, beta):
    raise NotImplementedError("write your pallas kernel here")


# speedup vs baseline: 3.7756x; 3.7756x over previous
import functools

import jax
import jax.numpy as jnp
from jax import lax
from jax.experimental import pallas as pl
from jax.experimental.pallas import tpu as pltpu
from jax.experimental.pallas import tpu_sc as plsc

N = 10000
D = 128
E = 320000

NC = 2
NS = 16
NW = NC * NS

B = 128
NB = 80
CH = 8
NCH = NB // CH
EW = B * NB
EPAD = NW * EW
NPAD = 10112
RPT = NPAD // NS
WPT = N // NS
NLB = 5


ROT = 624
TAIL = N - NS * ROT


def _idx_lists():
    r = jnp.arange(NLB * B, dtype=jnp.int32)
    base = jnp.arange(NS, dtype=jnp.int32)[:, None]
    init_idx = base * RPT + jnp.minimum(r, RPT - 1)[None, :]
    main = base * ROT + jnp.minimum(r, ROT - 1)[None, :]
    tail = (NS * ROT + (jnp.arange(B, dtype=jnp.int32) % TAIL))[None, :]
    wo_g = jnp.concatenate([main, jnp.tile(tail, (NS, 1))], axis=1)
    return init_idx.reshape(NS, NLB, B), wo_g.reshape(NS, NLB + 1, B)


def _sc_aggregate(x, src_p, dst_p, z2d, zdeg, ones8, init_idx, wo_g):
    mesh = plsc.VectorSubcoreMesh(core_axis_name="c", subcore_axis_name="s")

    @functools.partial(
        pl.kernel,
        out_type=(
            jax.ShapeDtypeStruct((NC * N, D), jnp.float32),
            jax.ShapeDtypeStruct((NC * N,), jnp.float32),
        ),
        mesh=mesh,
        scratch_types=[
            pltpu.VMEM((CH, B), jnp.int32),
            pltpu.VMEM((CH, B), jnp.int32),
            pltpu.VMEM((B, D), jnp.float32),
            pltpu.VMEM((B,), jnp.float32),
            pltpu.VMEM_SHARED((NPAD, D), jnp.float32),
            pltpu.VMEM_SHARED((NPAD,), jnp.float32),
            pltpu.SemaphoreType.DMA,
        ],
    )
    def k(x_h, src_h, dst_h, z2d_h, zdeg_h, ones_h, ii_h, wg_h,
          agg_o, deg_o, src_v, dst_v, rows_v, ones_v, agg_s, deg_s, sem):
        c = lax.axis_index("c")
        s = lax.axis_index("s")
        wid = c * NS + s
        pltpu.sync_copy(z2d_h, rows_v)
        pltpu.sync_copy(zdeg_h, ones_v)
        pltpu.sync_copy(ii_h.at[s], src_v.at[pl.ds(0, NLB)])
        for b in range(NLB):
            pltpu.sync_copy(rows_v, agg_s.at[src_v.at[b]])
            pltpu.sync_copy(ones_v, deg_s.at[src_v.at[b]])
        pltpu.sync_copy(ones_h, ones_v)
        plsc.subcore_barrier()

        def chunk(t, carry):
            pltpu.sync_copy(src_h.at[wid * NCH + t], src_v)
            pltpu.sync_copy(dst_h.at[wid * NCH + t], dst_v)

            for j in range(CH):
                pltpu.async_copy(x_h.at[src_v.at[j]], rows_v, sem).wait()
                pltpu.sync_copy(rows_v, agg_s.at[dst_v.at[j]], add=True)
                pltpu.sync_copy(ones_v, deg_s.at[dst_v.at[j]], add=True)
            return carry

        lax.fori_loop(0, NCH, chunk, 0)
        plsc.subcore_barrier()

        pltpu.sync_copy(wg_h.at[s], src_v.at[pl.ds(0, NLB + 1)])
        hb = c * N + s * ROT
        for b, (off, ln) in enumerate(((0, 128), (128, 128), (256, 128),
                                       (384, 128), (512, 112))):
            pltpu.sync_copy(agg_s.at[src_v.at[b]], rows_v)
            pltpu.sync_copy(rows_v.at[pl.ds(0, ln)],
                            agg_o.at[pl.ds(hb + off, ln)])
            pltpu.sync_copy(deg_s.at[src_v.at[b]], ones_v)
            pltpu.sync_copy(ones_v.at[pl.ds(0, ln)],
                            deg_o.at[pl.ds(hb + off, ln)])
        pltpu.sync_copy(agg_s.at[src_v.at[NLB]], rows_v)
        pltpu.sync_copy(rows_v.at[pl.ds(0, TAIL)],
                        agg_o.at[pl.ds(c * N + NS * ROT, TAIL)])
        pltpu.sync_copy(deg_s.at[src_v.at[NLB]], ones_v)
        pltpu.sync_copy(ones_v.at[pl.ds(0, TAIL)],
                        deg_o.at[pl.ds(c * N + NS * ROT, TAIL)])

    return k(x, src_p, dst_p, z2d, zdeg, ones8, init_idx, wo_g)


def _tc_dense(agg2, deg2, x, W_l, b_l, W_r, gamma, beta):
    BR = 400
    G = N // BR

    def body(aA, aB, dA, dB, xr, wl, wr, blr, gr, br, o):
        deg = dA[0][:, 0:1] + dB[0][:, 0:1]
        deg = jnp.maximum(deg, 1.0)
        agg = (aA[0] + aB[0]) / deg
        xb = xr[...]
        acc = lax.dot_general(agg, wl[...], (((1,), (1,)), ((), ())),
                              preferred_element_type=jnp.float32)
        acc = acc + lax.dot_general(xb, wr[...], (((1,), (1,)), ((), ())),
                                    preferred_element_type=jnp.float32)
        h = acc + blr[...] + xb
        h = jnp.maximum(h, 0.0)
        mu = jnp.mean(h, axis=-1, keepdims=True)
        hc = h - mu
        var = jnp.mean(hc * hc, axis=-1, keepdims=True)
        o[...] = hc * lax.rsqrt(var + 1e-5) * gr[...] + br[...]

    slabA = pl.BlockSpec((1, BR, D), lambda i: (0, i, 0))
    slabB = pl.BlockSpec((1, BR, D), lambda i: (1, i, 0))
    slabdA = pl.BlockSpec((1, BR, 1), lambda i: (0, i, 0))
    slabdB = pl.BlockSpec((1, BR, 1), lambda i: (1, i, 0))
    row = pl.BlockSpec((BR, D), lambda i: (i, 0))
    full = pl.BlockSpec((D, D), lambda i: (0, 0))
    vec = pl.BlockSpec((1, D), lambda i: (0, 0))
    return pl.pallas_call(
        body,
        grid=(G,),
        in_specs=[slabA, slabB, slabdA, slabdB, row, full, full, vec, vec, vec],
        out_specs=row,
        out_shape=jax.ShapeDtypeStruct((N, D), jnp.float32),
    )(agg2, agg2, deg2, deg2, x, W_l, W_r,
      b_l.reshape(1, D), gamma.reshape(1, D), beta.reshape(1, D))


def kernel(x, edge_index, W_l, b_l, W_r, gamma, beta):
    src = edge_index[0].astype(jnp.int32)
    dst = edge_index[1].astype(jnp.int32)
    src_p = jnp.concatenate(
        [src, jnp.zeros((EPAD - E,), jnp.int32)]).reshape(NW * NCH, CH, B)
    dst_p = jnp.concatenate(
        [dst, jnp.full((EPAD - E,), N, jnp.int32)]).reshape(NW * NCH, CH, B)
    z2d = jnp.zeros((B, D), jnp.float32)
    zdeg = jnp.zeros((B,), jnp.float32)
    ones8 = jnp.ones((B,), jnp.float32)
    init_idx, wo_g = _idx_lists()
    aggf, degf = _sc_aggregate(x, src_p, dst_p, z2d, zdeg, ones8,
                               init_idx, wo_g)
    agg2 = aggf.reshape(NC, N, D)
    deg2 = degf.reshape(NC, N, 1)
    return _tc_dense(agg2, deg2, x, W_l, b_l, W_r, gamma, beta)


# speedup vs baseline: 4.1132x; 1.0894x over previous
import functools

import jax
import jax.numpy as jnp
from jax import lax
from jax.experimental import pallas as pl
from jax.experimental.pallas import tpu as pltpu
from jax.experimental.pallas import tpu_sc as plsc

N = 10000
D = 128
E = 320000

NC = 2
NS = 16
NW = NC * NS

B = 128
NB = 80
CH = 8
NCH = NB // CH
EW = B * NB
EPAD = NW * EW
NPAD = 10112
RPT = NPAD // NS
WPT = N // NS
NLB = 5


ROT = 624
TAIL = N - NS * ROT


def _idx_lists():
    r = jnp.arange(NLB * B, dtype=jnp.int32)
    base = jnp.arange(NS, dtype=jnp.int32)[:, None]
    init_idx = base * RPT + jnp.minimum(r, RPT - 1)[None, :]
    main = base * ROT + jnp.minimum(r, ROT - 1)[None, :]
    tail = (NS * ROT + (jnp.arange(B, dtype=jnp.int32) % TAIL))[None, :]
    wo_g = jnp.concatenate([main, jnp.tile(tail, (NS, 1))], axis=1)
    return init_idx.reshape(NS, NLB, B), wo_g.reshape(NS, NLB + 1, B)


def _sc_aggregate(x, src_p, dst_p, z2d, zdeg, ones8, init_idx, wo_g):
    mesh = plsc.VectorSubcoreMesh(core_axis_name="c", subcore_axis_name="s")

    @functools.partial(
        pl.kernel,
        out_type=(
            jax.ShapeDtypeStruct((NC * N, D), jnp.float32),
            jax.ShapeDtypeStruct((NC * N,), jnp.float32),
        ),
        mesh=mesh,
        scratch_types=[
            pltpu.VMEM((CH, B), jnp.int32),
            pltpu.VMEM((CH, B), jnp.int32),
            pltpu.VMEM((B, D), jnp.float32),
            pltpu.VMEM((B, D), jnp.float32),
            pltpu.VMEM((B,), jnp.float32),
            pltpu.VMEM_SHARED((NPAD, D), jnp.float32),
            pltpu.VMEM_SHARED((NPAD,), jnp.float32),
            pltpu.SemaphoreType.DMA,
            pltpu.SemaphoreType.DMA,
        ],
    )
    def k(x_h, src_h, dst_h, z2d_h, zdeg_h, ones_h, ii_h, wg_h,
          agg_o, deg_o, src_v, dst_v, rows_v, rows2_v, ones_v, agg_s, deg_s,
          sem, sem2):
        c = lax.axis_index("c")
        s = lax.axis_index("s")
        wid = c * NS + s
        pltpu.sync_copy(z2d_h, rows_v)
        pltpu.sync_copy(zdeg_h, ones_v)
        pltpu.sync_copy(ii_h.at[s], src_v.at[pl.ds(0, NLB)])
        for b in range(NLB):
            pltpu.sync_copy(rows_v, agg_s.at[src_v.at[b]])
            pltpu.sync_copy(ones_v, deg_s.at[src_v.at[b]])
        pltpu.sync_copy(ones_h, ones_v)
        plsc.subcore_barrier()

        bufs = (rows_v, rows2_v)
        sems = (sem, sem2)

        def chunk(t, carry):
            pltpu.sync_copy(src_h.at[wid * NCH + t], src_v)
            pltpu.sync_copy(dst_h.at[wid * NCH + t], dst_v)

            cp = [None, None]
            cp[0] = pltpu.async_copy(x_h.at[src_v.at[0]], bufs[0], sems[0])
            for j in range(CH):
                if j + 1 < CH:
                    p = (j + 1) % 2
                    cp[p] = pltpu.async_copy(x_h.at[src_v.at[j + 1]],
                                             bufs[p], sems[p])
                cp[j % 2].wait()
                pltpu.sync_copy(bufs[j % 2], agg_s.at[dst_v.at[j]], add=True)
                pltpu.sync_copy(ones_v, deg_s.at[dst_v.at[j]], add=True)
            return carry

        lax.fori_loop(0, NCH, chunk, 0)
        plsc.subcore_barrier()

        pltpu.sync_copy(wg_h.at[s], src_v.at[pl.ds(0, NLB + 1)])
        hb = c * N + s * ROT
        for b, (off, ln) in enumerate(((0, 128), (128, 128), (256, 128),
                                       (384, 128), (512, 112))):
            pltpu.sync_copy(agg_s.at[src_v.at[b]], rows_v)
            pltpu.sync_copy(rows_v.at[pl.ds(0, ln)],
                            agg_o.at[pl.ds(hb + off, ln)])
            pltpu.sync_copy(deg_s.at[src_v.at[b]], ones_v)
            pltpu.sync_copy(ones_v.at[pl.ds(0, ln)],
                            deg_o.at[pl.ds(hb + off, ln)])
        pltpu.sync_copy(agg_s.at[src_v.at[NLB]], rows_v)
        pltpu.sync_copy(rows_v.at[pl.ds(0, TAIL)],
                        agg_o.at[pl.ds(c * N + NS * ROT, TAIL)])
        pltpu.sync_copy(deg_s.at[src_v.at[NLB]], ones_v)
        pltpu.sync_copy(ones_v.at[pl.ds(0, TAIL)],
                        deg_o.at[pl.ds(c * N + NS * ROT, TAIL)])

    return k(x, src_p, dst_p, z2d, zdeg, ones8, init_idx, wo_g)


def _tc_dense(agg2, deg2, x, W_l, b_l, W_r, gamma, beta):
    BR = 400
    G = N // BR

    def body(aA, aB, dA, dB, xr, wl, wr, blr, gr, br, o):
        deg = dA[0][:, 0:1] + dB[0][:, 0:1]
        deg = jnp.maximum(deg, 1.0)
        agg = (aA[0] + aB[0]) / deg
        xb = xr[...]
        acc = lax.dot_general(agg, wl[...], (((1,), (1,)), ((), ())),
                              preferred_element_type=jnp.float32)
        acc = acc + lax.dot_general(xb, wr[...], (((1,), (1,)), ((), ())),
                                    preferred_element_type=jnp.float32)
        h = acc + blr[...] + xb
        h = jnp.maximum(h, 0.0)
        mu = jnp.mean(h, axis=-1, keepdims=True)
        hc = h - mu
        var = jnp.mean(hc * hc, axis=-1, keepdims=True)
        o[...] = hc * lax.rsqrt(var + 1e-5) * gr[...] + br[...]

    slabA = pl.BlockSpec((1, BR, D), lambda i: (0, i, 0))
    slabB = pl.BlockSpec((1, BR, D), lambda i: (1, i, 0))
    slabdA = pl.BlockSpec((1, BR, 1), lambda i: (0, i, 0))
    slabdB = pl.BlockSpec((1, BR, 1), lambda i: (1, i, 0))
    row = pl.BlockSpec((BR, D), lambda i: (i, 0))
    full = pl.BlockSpec((D, D), lambda i: (0, 0))
    vec = pl.BlockSpec((1, D), lambda i: (0, 0))
    return pl.pallas_call(
        body,
        grid=(G,),
        in_specs=[slabA, slabB, slabdA, slabdB, row, full, full, vec, vec, vec],
        out_specs=row,
        out_shape=jax.ShapeDtypeStruct((N, D), jnp.float32),
    )(agg2, agg2, deg2, deg2, x, W_l, W_r,
      b_l.reshape(1, D), gamma.reshape(1, D), beta.reshape(1, D))


def kernel(x, edge_index, W_l, b_l, W_r, gamma, beta):
    src = edge_index[0].astype(jnp.int32)
    dst = edge_index[1].astype(jnp.int32)
    src_p = jnp.concatenate(
        [src, jnp.zeros((EPAD - E,), jnp.int32)]).reshape(NW * NCH, CH, B)
    dst_p = jnp.concatenate(
        [dst, jnp.full((EPAD - E,), N, jnp.int32)]).reshape(NW * NCH, CH, B)
    z2d = jnp.zeros((B, D), jnp.float32)
    zdeg = jnp.zeros((B,), jnp.float32)
    ones8 = jnp.ones((B,), jnp.float32)
    init_idx, wo_g = _idx_lists()
    aggf, degf = _sc_aggregate(x, src_p, dst_p, z2d, zdeg, ones8,
                               init_idx, wo_g)
    agg2 = aggf.reshape(NC, N, D)
    deg2 = degf.reshape(NC, N, 1)
    return _tc_dense(agg2, deg2, x, W_l, b_l, W_r, gamma, beta)


# speedup vs baseline: 4.1197x; 1.0016x over previous
import functools

import jax
import jax.numpy as jnp
from jax import lax
from jax.experimental import pallas as pl
from jax.experimental.pallas import tpu as pltpu
from jax.experimental.pallas import tpu_sc as plsc

N = 10000
D = 128
E = 320000

NC = 2
NS = 16
NW = NC * NS

B = 128
NB = 80
CH = 8
NCH = NB // CH
EW = B * NB
EPAD = NW * EW
NPAD = 10112
RPT = NPAD // NS
WPT = N // NS
NLB = 5


ROT = 624
TAIL = N - NS * ROT


def _idx_lists():
    r = jnp.arange(NLB * B, dtype=jnp.int32)
    base = jnp.arange(NS, dtype=jnp.int32)[:, None]
    init_idx = base * RPT + jnp.minimum(r, RPT - 1)[None, :]
    main = base * ROT + jnp.minimum(r, ROT - 1)[None, :]
    tail = (NS * ROT + (jnp.arange(B, dtype=jnp.int32) % TAIL))[None, :]
    wo_g = jnp.concatenate([main, jnp.tile(tail, (NS, 1))], axis=1)
    return init_idx.reshape(NS, NLB, B), wo_g.reshape(NS, NLB + 1, B)


def _sc_aggregate(x, src_p, dst_p, z2d, zdeg, ones8, init_idx, wo_g):
    mesh = plsc.VectorSubcoreMesh(core_axis_name="c", subcore_axis_name="s")

    @functools.partial(
        pl.kernel,
        out_type=(
            jax.ShapeDtypeStruct((NC * N, D), jnp.float32),
            jax.ShapeDtypeStruct((NC * N,), jnp.float32),
        ),
        mesh=mesh,
        scratch_types=[
            pltpu.VMEM((CH, B), jnp.int32),
            pltpu.VMEM((CH, B), jnp.int32),
            pltpu.VMEM((B, D), jnp.float32),
            pltpu.VMEM((B, D), jnp.float32),
            pltpu.VMEM((B,), jnp.float32),
            pltpu.VMEM_SHARED((NPAD, D), jnp.float32),
            pltpu.VMEM_SHARED((NPAD,), jnp.float32),
            pltpu.SemaphoreType.DMA,
            pltpu.SemaphoreType.DMA,
            pltpu.SemaphoreType.DMA,
        ],
    )
    def k(x_h, src_h, dst_h, z2d_h, zdeg_h, ones_h, ii_h, wg_h,
          agg_o, deg_o, src_v, dst_v, rows_v, rows2_v, ones_v, agg_s, deg_s,
          sem, sem2, sem3):
        c = lax.axis_index("c")
        s = lax.axis_index("s")
        wid = c * NS + s
        pltpu.sync_copy(z2d_h, rows_v)
        pltpu.sync_copy(zdeg_h, ones_v)
        pltpu.sync_copy(ii_h.at[s], src_v.at[pl.ds(0, NLB)])
        for b in range(NLB):
            pltpu.sync_copy(rows_v, agg_s.at[src_v.at[b]])
            pltpu.sync_copy(ones_v, deg_s.at[src_v.at[b]])
        pltpu.sync_copy(ones_h, ones_v)
        plsc.subcore_barrier()

        bufs = (rows_v, rows2_v)
        sems = (sem, sem2)

        def chunk(t, carry):
            pltpu.sync_copy(src_h.at[wid * NCH + t], src_v)
            pltpu.sync_copy(dst_h.at[wid * NCH + t], dst_v)

            cp = [None, None]
            cp[0] = pltpu.async_copy(x_h.at[src_v.at[0]], bufs[0], sems[0])
            for j in range(CH):
                if j + 1 < CH:
                    p = (j + 1) % 2
                    cp[p] = pltpu.async_copy(x_h.at[src_v.at[j + 1]],
                                             bufs[p], sems[p])
                cp[j % 2].wait()
                dcp = pltpu.async_copy(ones_v, deg_s.at[dst_v.at[j]], sem3,
                                       add=True)
                pltpu.sync_copy(bufs[j % 2], agg_s.at[dst_v.at[j]], add=True)
                dcp.wait()
            return carry

        lax.fori_loop(0, NCH, chunk, 0)
        plsc.subcore_barrier()

        pltpu.sync_copy(wg_h.at[s], src_v.at[pl.ds(0, NLB + 1)])
        hb = c * N + s * ROT
        for b, (off, ln) in enumerate(((0, 128), (128, 128), (256, 128),
                                       (384, 128), (512, 112))):
            pltpu.sync_copy(agg_s.at[src_v.at[b]], rows_v)
            pltpu.sync_copy(rows_v.at[pl.ds(0, ln)],
                            agg_o.at[pl.ds(hb + off, ln)])
            pltpu.sync_copy(deg_s.at[src_v.at[b]], ones_v)
            pltpu.sync_copy(ones_v.at[pl.ds(0, ln)],
                            deg_o.at[pl.ds(hb + off, ln)])
        pltpu.sync_copy(agg_s.at[src_v.at[NLB]], rows_v)
        pltpu.sync_copy(rows_v.at[pl.ds(0, TAIL)],
                        agg_o.at[pl.ds(c * N + NS * ROT, TAIL)])
        pltpu.sync_copy(deg_s.at[src_v.at[NLB]], ones_v)
        pltpu.sync_copy(ones_v.at[pl.ds(0, TAIL)],
                        deg_o.at[pl.ds(c * N + NS * ROT, TAIL)])

    return k(x, src_p, dst_p, z2d, zdeg, ones8, init_idx, wo_g)


def _tc_dense(agg2, deg2, x, W_l, b_l, W_r, gamma, beta):
    BR = 400
    G = N // BR

    def body(aA, aB, dA, dB, xr, wl, wr, blr, gr, br, o):
        deg = dA[0][:, 0:1] + dB[0][:, 0:1]
        deg = jnp.maximum(deg, 1.0)
        agg = (aA[0] + aB[0]) / deg
        xb = xr[...]
        acc = lax.dot_general(agg, wl[...], (((1,), (1,)), ((), ())),
                              preferred_element_type=jnp.float32)
        acc = acc + lax.dot_general(xb, wr[...], (((1,), (1,)), ((), ())),
                                    preferred_element_type=jnp.float32)
        h = acc + blr[...] + xb
        h = jnp.maximum(h, 0.0)
        mu = jnp.mean(h, axis=-1, keepdims=True)
        hc = h - mu
        var = jnp.mean(hc * hc, axis=-1, keepdims=True)
        o[...] = hc * lax.rsqrt(var + 1e-5) * gr[...] + br[...]

    slabA = pl.BlockSpec((1, BR, D), lambda i: (0, i, 0))
    slabB = pl.BlockSpec((1, BR, D), lambda i: (1, i, 0))
    slabdA = pl.BlockSpec((1, BR, 1), lambda i: (0, i, 0))
    slabdB = pl.BlockSpec((1, BR, 1), lambda i: (1, i, 0))
    row = pl.BlockSpec((BR, D), lambda i: (i, 0))
    full = pl.BlockSpec((D, D), lambda i: (0, 0))
    vec = pl.BlockSpec((1, D), lambda i: (0, 0))
    return pl.pallas_call(
        body,
        grid=(G,),
        in_specs=[slabA, slabB, slabdA, slabdB, row, full, full, vec, vec, vec],
        out_specs=row,
        out_shape=jax.ShapeDtypeStruct((N, D), jnp.float32),
    )(agg2, agg2, deg2, deg2, x, W_l, W_r,
      b_l.reshape(1, D), gamma.reshape(1, D), beta.reshape(1, D))


def kernel(x, edge_index, W_l, b_l, W_r, gamma, beta):
    src = edge_index[0].astype(jnp.int32)
    dst = edge_index[1].astype(jnp.int32)
    src_p = jnp.concatenate(
        [src, jnp.zeros((EPAD - E,), jnp.int32)]).reshape(NW * NCH, CH, B)
    dst_p = jnp.concatenate(
        [dst, jnp.full((EPAD - E,), N, jnp.int32)]).reshape(NW * NCH, CH, B)
    z2d = jnp.zeros((B, D), jnp.float32)
    zdeg = jnp.zeros((B,), jnp.float32)
    ones8 = jnp.ones((B,), jnp.float32)
    init_idx, wo_g = _idx_lists()
    aggf, degf = _sc_aggregate(x, src_p, dst_p, z2d, zdeg, ones8,
                               init_idx, wo_g)
    agg2 = aggf.reshape(NC, N, D)
    deg2 = degf.reshape(NC, N, 1)
    return _tc_dense(agg2, deg2, x, W_l, b_l, W_r, gamma, beta)


# speedup vs baseline: 4.1222x; 1.0006x over previous
import functools

import jax
import jax.numpy as jnp
from jax import lax
from jax.experimental import pallas as pl
from jax.experimental.pallas import tpu as pltpu
from jax.experimental.pallas import tpu_sc as plsc

N = 10000
D = 128
E = 320000

NC = 2
NS = 16
NW = NC * NS

B = 128
NB = 80
CH = 8
NCH = NB // CH
EW = B * NB
EPAD = NW * EW
NPAD = 10112
RPT = NPAD // NS
WPT = N // NS
NLB = 5


ROT = 624
TAIL = N - NS * ROT


def _idx_lists():
    r = jnp.arange(NLB * B, dtype=jnp.int32)
    base = jnp.arange(NS, dtype=jnp.int32)[:, None]
    init_idx = base * RPT + jnp.minimum(r, RPT - 1)[None, :]
    main = base * ROT + jnp.minimum(r, ROT - 1)[None, :]
    tail = (NS * ROT + (jnp.arange(B, dtype=jnp.int32) % TAIL))[None, :]
    wo_g = jnp.concatenate([main, jnp.tile(tail, (NS, 1))], axis=1)
    return init_idx.reshape(NS, NLB, B), wo_g.reshape(NS, NLB + 1, B)


def _sc_aggregate(x, src_p, dst_p, z2d, zdeg, ones8, init_idx, wo_g):
    mesh = plsc.VectorSubcoreMesh(core_axis_name="c", subcore_axis_name="s")

    @functools.partial(
        pl.kernel,
        out_type=(
            jax.ShapeDtypeStruct((NC * N, D), jnp.float32),
            jax.ShapeDtypeStruct((NC * N,), jnp.float32),
        ),
        mesh=mesh,
        scratch_types=[
            pltpu.VMEM((CH, B), jnp.int32),
            pltpu.VMEM((CH, B), jnp.int32),
            pltpu.VMEM((B, D), jnp.float32),
            pltpu.VMEM((B, D), jnp.float32),
            pltpu.VMEM((B,), jnp.float32),
            pltpu.VMEM_SHARED((NPAD, D), jnp.float32),
            pltpu.VMEM_SHARED((NPAD,), jnp.float32),
            pltpu.SemaphoreType.DMA,
            pltpu.SemaphoreType.DMA,
            pltpu.SemaphoreType.DMA,
            pltpu.SemaphoreType.DMA,
            pltpu.SemaphoreType.DMA,
        ],
    )
    def k(x_h, src_h, dst_h, z2d_h, zdeg_h, ones_h, ii_h, wg_h,
          agg_o, deg_o, src_v, dst_v, rows_v, rows2_v, ones_v, agg_s, deg_s,
          sem, sem2, sem3, semsa, semsb):
        c = lax.axis_index("c")
        s = lax.axis_index("s")
        wid = c * NS + s
        pltpu.sync_copy(z2d_h, rows_v)
        pltpu.sync_copy(zdeg_h, ones_v)
        pltpu.sync_copy(ii_h.at[s], src_v.at[pl.ds(0, NLB)])
        for b in range(NLB):
            pltpu.sync_copy(rows_v, agg_s.at[src_v.at[b]])
            pltpu.sync_copy(ones_v, deg_s.at[src_v.at[b]])
        pltpu.sync_copy(ones_h, ones_v)
        plsc.subcore_barrier()

        bufs = (rows_v, rows2_v)
        sems = (sem, sem2)
        ssems = (semsa, semsb)

        def chunk(t, carry):
            pltpu.sync_copy(src_h.at[wid * NCH + t], src_v)
            pltpu.sync_copy(dst_h.at[wid * NCH + t], dst_v)

            cp = [None, None]
            scp = [None, None]
            cp[0] = pltpu.async_copy(x_h.at[src_v.at[0]], bufs[0], sems[0])
            for j in range(CH):
                if j + 1 < CH:
                    p = (j + 1) % 2
                    if scp[p] is not None:
                        scp[p].wait()
                    cp[p] = pltpu.async_copy(x_h.at[src_v.at[j + 1]],
                                             bufs[p], sems[p])
                cp[j % 2].wait()
                dcp = pltpu.async_copy(ones_v, deg_s.at[dst_v.at[j]], sem3,
                                       add=True)
                scp[j % 2] = pltpu.async_copy(bufs[j % 2],
                                              agg_s.at[dst_v.at[j]],
                                              ssems[j % 2], add=True)
                dcp.wait()
            for p in (0, 1):
                if scp[p] is not None:
                    scp[p].wait()
            return carry

        lax.fori_loop(0, NCH, chunk, 0)
        plsc.subcore_barrier()

        pltpu.sync_copy(wg_h.at[s], src_v.at[pl.ds(0, NLB + 1)])
        hb = c * N + s * ROT
        for b, (off, ln) in enumerate(((0, 128), (128, 128), (256, 128),
                                       (384, 128), (512, 112))):
            pltpu.sync_copy(agg_s.at[src_v.at[b]], rows_v)
            pltpu.sync_copy(rows_v.at[pl.ds(0, ln)],
                            agg_o.at[pl.ds(hb + off, ln)])
            pltpu.sync_copy(deg_s.at[src_v.at[b]], ones_v)
            pltpu.sync_copy(ones_v.at[pl.ds(0, ln)],
                            deg_o.at[pl.ds(hb + off, ln)])
        pltpu.sync_copy(agg_s.at[src_v.at[NLB]], rows_v)
        pltpu.sync_copy(rows_v.at[pl.ds(0, TAIL)],
                        agg_o.at[pl.ds(c * N + NS * ROT, TAIL)])
        pltpu.sync_copy(deg_s.at[src_v.at[NLB]], ones_v)
        pltpu.sync_copy(ones_v.at[pl.ds(0, TAIL)],
                        deg_o.at[pl.ds(c * N + NS * ROT, TAIL)])

    return k(x, src_p, dst_p, z2d, zdeg, ones8, init_idx, wo_g)


def _tc_dense(agg2, deg2, x, W_l, b_l, W_r, gamma, beta):
    BR = 400
    G = N // BR

    def body(aA, aB, dA, dB, xr, wl, wr, blr, gr, br, o):
        deg = dA[0][:, 0:1] + dB[0][:, 0:1]
        deg = jnp.maximum(deg, 1.0)
        agg = (aA[0] + aB[0]) / deg
        xb = xr[...]
        acc = lax.dot_general(agg, wl[...], (((1,), (1,)), ((), ())),
                              preferred_element_type=jnp.float32)
        acc = acc + lax.dot_general(xb, wr[...], (((1,), (1,)), ((), ())),
                                    preferred_element_type=jnp.float32)
        h = acc + blr[...] + xb
        h = jnp.maximum(h, 0.0)
        mu = jnp.mean(h, axis=-1, keepdims=True)
        hc = h - mu
        var = jnp.mean(hc * hc, axis=-1, keepdims=True)
        o[...] = hc * lax.rsqrt(var + 1e-5) * gr[...] + br[...]

    slabA = pl.BlockSpec((1, BR, D), lambda i: (0, i, 0))
    slabB = pl.BlockSpec((1, BR, D), lambda i: (1, i, 0))
    slabdA = pl.BlockSpec((1, BR, 1), lambda i: (0, i, 0))
    slabdB = pl.BlockSpec((1, BR, 1), lambda i: (1, i, 0))
    row = pl.BlockSpec((BR, D), lambda i: (i, 0))
    full = pl.BlockSpec((D, D), lambda i: (0, 0))
    vec = pl.BlockSpec((1, D), lambda i: (0, 0))
    return pl.pallas_call(
        body,
        grid=(G,),
        in_specs=[slabA, slabB, slabdA, slabdB, row, full, full, vec, vec, vec],
        out_specs=row,
        out_shape=jax.ShapeDtypeStruct((N, D), jnp.float32),
    )(agg2, agg2, deg2, deg2, x, W_l, W_r,
      b_l.reshape(1, D), gamma.reshape(1, D), beta.reshape(1, D))


def kernel(x, edge_index, W_l, b_l, W_r, gamma, beta):
    src = edge_index[0].astype(jnp.int32)
    dst = edge_index[1].astype(jnp.int32)
    src_p = jnp.concatenate(
        [src, jnp.zeros((EPAD - E,), jnp.int32)]).reshape(NW * NCH, CH, B)
    dst_p = jnp.concatenate(
        [dst, jnp.full((EPAD - E,), N, jnp.int32)]).reshape(NW * NCH, CH, B)
    z2d = jnp.zeros((B, D), jnp.float32)
    zdeg = jnp.zeros((B,), jnp.float32)
    ones8 = jnp.ones((B,), jnp.float32)
    init_idx, wo_g = _idx_lists()
    aggf, degf = _sc_aggregate(x, src_p, dst_p, z2d, zdeg, ones8,
                               init_idx, wo_g)
    agg2 = aggf.reshape(NC, N, D)
    deg2 = degf.reshape(NC, N, 1)
    return _tc_dense(agg2, deg2, x, W_l, b_l, W_r, gamma, beta)
